# baseline (speedup 1.0000x reference)
import jax
import jax.numpy as jnp
from jax.experimental import pallas as pl
from jax.experimental.pallas import tpu as pltpu

_NUM_EXPERTS = 8
_D = 768
_HEADS = 12
_DEPTH = 64
_LANE = 128
_BT = 512
_NEG = -1e30


def _cast_body(we_ref, wq_ref, wk_ref, wv_ref, wo_ref,
               wall_ref, wqb_ref, wkb_ref, wvb_ref, wob_ref):
    i = pl.program_id(0)
    wall_ref[...] = we_ref[0].astype(jnp.bfloat16)

    @pl.when(i == 0)
    def _():
        wqb_ref[...] = wq_ref[...].astype(jnp.bfloat16)
        wkb_ref[...] = wk_ref[...].astype(jnp.bfloat16)
        wvb_ref[...] = wv_ref[...].astype(jnp.bfloat16)
        wob_ref[...] = wo_ref[...].astype(jnp.bfloat16)


def _fused_body(x_ref, wr_ref, br_ref, wall_ref, be_ref, wq_ref, bq_ref,
                wk_ref, bk_ref, wv_ref, bv_ref, wo_ref, bo_ref,
                hmap_ref, hmapt_ref, o_ref):
    x = x_ref[...]

    logits = jnp.dot(x, wr_ref[...], preferred_element_type=jnp.float32)
    logits = logits + br_ref[...]
    m = jnp.max(logits, axis=-1, keepdims=True)
    e = jnp.exp(logits - m)
    probs = e / jnp.sum(e, axis=-1, keepdims=True)

    cols = jax.lax.broadcasted_iota(jnp.int32, probs.shape, 1)
    p1 = jnp.max(probs, axis=-1, keepdims=True)
    i1 = jnp.min(jnp.where(probs >= p1, cols, _LANE), axis=-1, keepdims=True)
    probs_m = jnp.where(cols == i1, -1.0, probs)
    p2 = jnp.max(probs_m, axis=-1, keepdims=True)
    i2 = jnp.min(jnp.where(probs_m >= p2, cols, _LANE), axis=-1, keepdims=True)
    sel = (cols == i1) | (cols == i2)
    w = jnp.where(sel, probs, 0.0)

    xb = x.astype(jnp.bfloat16)
    y = jnp.dot(xb, wall_ref[...], preferred_element_type=jnp.float32)
    z = jnp.maximum(y + be_ref[...], 0.0)
    c = [z[:, i * _D:(i + 1) * _D] * w[:, i:i + 1] for i in range(_NUM_EXPERTS)]
    combined = ((c[0] + c[1]) + (c[2] + c[3])) + ((c[4] + c[5]) + (c[6] + c[7]))

    cb = combined.astype(jnp.bfloat16)
    q = jnp.dot(cb, wq_ref[...], preferred_element_type=jnp.float32) + bq_ref[...]
    k = jnp.dot(cb, wk_ref[...], preferred_element_type=jnp.float32) + bk_ref[...]
    v = jnp.dot(cb, wv_ref[...], preferred_element_type=jnp.float32) + bv_ref[...]
    hmap = hmap_ref[...]
    s = jnp.dot((q * k).astype(jnp.bfloat16), hmap,
                preferred_element_type=jnp.float32)
    s = s * (1.0 / jnp.sqrt(jnp.float32(_DEPTH)))
    s = jnp.where(jax.lax.broadcasted_iota(jnp.int32, s.shape, 1) < _HEADS,
                  s, _NEG)
    sm = jnp.max(s, axis=-1, keepdims=True)
    se = jnp.exp(s - sm)
    aw = se / jnp.sum(se, axis=-1, keepdims=True)
    wb = jnp.dot(aw.astype(jnp.bfloat16), hmapt_ref[...],
                 preferred_element_type=jnp.float32)
    out = jnp.dot((wb * v).astype(jnp.bfloat16), wo_ref[...],
                  preferred_element_type=jnp.float32)
    o_ref[...] = out + bo_ref[...]


@jax.jit
def kernel(inputs, Wr, br, We, be, Wq, bq, Wk, bk, Wv, bv, Wo, bo):
    n = inputs.shape[0]
    bf = jnp.bfloat16
    de = _D * _NUM_EXPERTS

    dxd = lambda: pl.BlockSpec((_D, _D), lambda i: (0, 0))
    wall, wqb, wkb, wvb, wob = pl.pallas_call(
        _cast_body,
        grid=(_NUM_EXPERTS,),
        in_specs=[pl.BlockSpec((1, _D, _D), lambda i: (i, 0, 0)),
                  dxd(), dxd(), dxd(), dxd()],
        out_specs=[pl.BlockSpec((_D, _D), lambda i: (0, i)),
                   dxd(), dxd(), dxd(), dxd()],
        out_shape=[jax.ShapeDtypeStruct((_D, de), bf),
                   jax.ShapeDtypeStruct((_D, _D), bf),
                   jax.ShapeDtypeStruct((_D, _D), bf),
                   jax.ShapeDtypeStruct((_D, _D), bf),
                   jax.ShapeDtypeStruct((_D, _D), bf)],
        compiler_params=pltpu.CompilerParams(
            dimension_semantics=("arbitrary",),
        ),
    )(We, Wq, Wk, Wv, Wo)

    wr_p = jnp.zeros((_D, _LANE), jnp.float32).at[:, :_NUM_EXPERTS].set(Wr)
    br_p = jnp.full((1, _LANE), _NEG, jnp.float32).at[0, :_NUM_EXPERTS].set(br)
    d_idx = jnp.arange(_D) // _DEPTH
    hmap = (d_idx[:, None] == jnp.arange(_LANE)[None, :]).astype(bf)
    hmapt = hmap.T
    be_flat = be.reshape(1, de)

    grid = (n // _BT,)
    full = lambda shape: pl.BlockSpec(shape, lambda i: (0,) * len(shape))
    out = pl.pallas_call(
        _fused_body,
        grid=grid,
        in_specs=[
            pl.BlockSpec((_BT, _D), lambda i: (i, 0)),
            full((_D, _LANE)),
            full((1, _LANE)),
            full((_D, de)),
            full((1, de)),
            full((_D, _D)), full((1, _D)),
            full((_D, _D)), full((1, _D)),
            full((_D, _D)), full((1, _D)),
            full((_D, _D)), full((1, _D)),
            full((_D, _LANE)),
            full((_LANE, _D)),
        ],
        out_specs=pl.BlockSpec((_BT, _D), lambda i: (i, 0)),
        out_shape=jax.ShapeDtypeStruct((n, _D), jnp.float32),
        compiler_params=pltpu.CompilerParams(
            dimension_semantics=("parallel",),
        ),
    )(inputs, wr_p, br_p, wall, be_flat,
      wqb, bq.reshape(1, _D), wkb, bk.reshape(1, _D),
      wvb, bv.reshape(1, _D), wob, bo.reshape(1, _D),
      hmap, hmapt)
    return out


# speedup vs baseline: 1.1449x; 1.1449x over previous
import jax
import jax.numpy as jnp
from jax.experimental import pallas as pl
from jax.experimental.pallas import tpu as pltpu

_NUM_EXPERTS = 8
_D = 768
_HEADS = 12
_DEPTH = 64
_LANE = 128
_BT = 512
_NEG = -1e30


def _fused_body(x_ref, wr_ref, br_ref, we_ref, be_ref, wq_ref, bq_ref,
                wk_ref, bk_ref, wv_ref, bv_ref, wo_ref, bo_ref, o_ref,
                wrp_ref, brp_ref, hmap_ref, hmapt_ref):
    i = pl.program_id(0)

    @pl.when(i == 0)
    def _prep():
        wrp_ref[...] = jnp.concatenate(
            [wr_ref[...], jnp.zeros((_D, _LANE - _NUM_EXPERTS), jnp.float32)],
            axis=1)
        brp_ref[...] = jnp.concatenate(
            [br_ref[...], jnp.full((1, _LANE - _NUM_EXPERTS), _NEG,
                                   jnp.float32)], axis=1)
        di = jax.lax.broadcasted_iota(jnp.int32, (_D, _LANE), 0) // _DEPTH
        hi = jax.lax.broadcasted_iota(jnp.int32, (_D, _LANE), 1)
        hmap_ref[...] = (di == hi).astype(jnp.float32)
        dit = jax.lax.broadcasted_iota(jnp.int32, (_LANE, _D), 1) // _DEPTH
        hit = jax.lax.broadcasted_iota(jnp.int32, (_LANE, _D), 0)
        hmapt_ref[...] = (dit == hit).astype(jnp.float32)

    x = x_ref[...]

    logits = jnp.dot(x, wrp_ref[...], preferred_element_type=jnp.float32)
    logits = logits + brp_ref[...]
    m = jnp.max(logits, axis=-1, keepdims=True)
    e = jnp.exp(logits - m)
    probs = e / jnp.sum(e, axis=-1, keepdims=True)

    cols = jax.lax.broadcasted_iota(jnp.int32, probs.shape, 1)
    p1 = jnp.max(probs, axis=-1, keepdims=True)
    i1 = jnp.min(jnp.where(probs >= p1, cols, _LANE), axis=-1, keepdims=True)
    probs_m = jnp.where(cols == i1, -1.0, probs)
    p2 = jnp.max(probs_m, axis=-1, keepdims=True)
    i2 = jnp.min(jnp.where(probs_m >= p2, cols, _LANE), axis=-1, keepdims=True)
    sel = (cols == i1) | (cols == i2)
    w = jnp.where(sel, probs, 0.0)

    combined = jnp.zeros((x.shape[0], _D), dtype=jnp.float32)
    for i_e in range(_NUM_EXPERTS):
        eo = jnp.dot(x, we_ref[i_e], preferred_element_type=jnp.float32)
        eo = jnp.maximum(eo + be_ref[i_e:i_e + 1, :], 0.0)
        combined = combined + eo * w[:, i_e:i_e + 1]

    q = jnp.dot(combined, wq_ref[...], preferred_element_type=jnp.float32) + bq_ref[...]
    k = jnp.dot(combined, wk_ref[...], preferred_element_type=jnp.float32) + bk_ref[...]
    v = jnp.dot(combined, wv_ref[...], preferred_element_type=jnp.float32) + bv_ref[...]
    s = jnp.dot(q * k, hmap_ref[...], preferred_element_type=jnp.float32)
    s = s * (1.0 / jnp.sqrt(jnp.float32(_DEPTH)))
    s = jnp.where(cols < _HEADS, s, _NEG)
    sm = jnp.max(s, axis=-1, keepdims=True)
    se = jnp.exp(s - sm)
    aw = se / jnp.sum(se, axis=-1, keepdims=True)
    wb = jnp.dot(aw, hmapt_ref[...], preferred_element_type=jnp.float32)
    out = jnp.dot(wb * v, wo_ref[...], preferred_element_type=jnp.float32)
    o_ref[...] = out + bo_ref[...]


@jax.jit
def kernel(inputs, Wr, br, We, be, Wq, bq, Wk, bk, Wv, bv, Wo, bo):
    n = inputs.shape[0]
    grid = (n // _BT,)
    full = lambda shape: pl.BlockSpec(shape, lambda i: (0,) * len(shape))
    out = pl.pallas_call(
        _fused_body,
        grid=grid,
        in_specs=[
            pl.BlockSpec((_BT, _D), lambda i: (i, 0)),
            full((_D, _NUM_EXPERTS)),
            full((1, _NUM_EXPERTS)),
            full((_NUM_EXPERTS, _D, _D)),
            full((_NUM_EXPERTS, _D)),
            full((_D, _D)), full((1, _D)),
            full((_D, _D)), full((1, _D)),
            full((_D, _D)), full((1, _D)),
            full((_D, _D)), full((1, _D)),
        ],
        out_specs=pl.BlockSpec((_BT, _D), lambda i: (i, 0)),
        out_shape=jax.ShapeDtypeStruct((n, _D), jnp.float32),
        scratch_shapes=[
            pltpu.VMEM((_D, _LANE), jnp.float32),
            pltpu.VMEM((1, _LANE), jnp.float32),
            pltpu.VMEM((_D, _LANE), jnp.float32),
            pltpu.VMEM((_LANE, _D), jnp.float32),
        ],
        compiler_params=pltpu.CompilerParams(
            dimension_semantics=("arbitrary",),
        ),
    )(inputs, Wr, br.reshape(1, _NUM_EXPERTS), We, be,
      Wq, bq.reshape(1, _D), Wk, bk.reshape(1, _D),
      Wv, bv.reshape(1, _D), Wo, bo.reshape(1, _D))
    return out
